# baseline (speedup 1.0000x reference)
import dataclasses

import jax
import jax.numpy as jnp
from jax import lax
from jax.experimental import pallas as pl
from jax.experimental.pallas import tpu as pltpu
from jax.experimental.pallas import tpu_sc as plsc

_cp = pltpu.CompilerParams()
if "needs_layout_passes" in pltpu.CompilerParams.__dataclass_fields__:
    _cp = dataclasses.replace(_cp, needs_layout_passes=False)

NC = 2
NS = 16
NW = NC * NS
LANES = 16

N_NODES = 10000
N_PAD = 10240
D = 128
E_TOTAL = 480000

DEG_CHUNK = 120
DEG_NCH = E_TOTAL // NS // DEG_CHUNK
EDGE_CHUNK = 120
EDGE_NCH = E_TOTAL // NW // EDGE_CHUNK
NQUAD = (EDGE_NCH - 1) // 4

ROWS_PER_TILE = N_PAD // NW
ROWS_PER_SCTILE = N_PAD // NS

_mesh = plsc.VectorSubcoreMesh(core_axis_name="c", subcore_axis_name="s")


def _zero16():
    return jnp.zeros((LANES,), jnp.float32)


def _fast_rsqrt(d):
    bits = plsc.bitcast(d, jnp.int32)
    bits = jnp.int32(0x5F3759DF) - (bits >> 1)
    r = plsc.bitcast(bits, jnp.float32)
    for _ in range(3):
        r = r * (1.5 - 0.5 * d * r * r)
    return jnp.where(d > 0.5, r, 0.0)


def _k0_body(col3_hbm, degp_out, deg_sp, ones_v, col2d_v, z_v, sem_d):
    c = lax.axis_index("c")
    s = lax.axis_index("s")
    w = s * NC + c

    pltpu.sync_copy(col3_hbm.at[w], col2d_v)

    @pl.loop(0, ROWS_PER_SCTILE // LANES)
    def _(i):
        z_v[pl.ds(i * LANES, LANES)] = _zero16()

    for i in range(128 // LANES):
        ones_v[pl.ds(i * LANES, LANES)] = jnp.ones((LANES,), jnp.float32)

    pltpu.sync_copy(z_v, deg_sp.at[pl.ds(s * ROWS_PER_SCTILE, ROWS_PER_SCTILE)])
    plsc.subcore_barrier()

    ones_src = ones_v.at[pl.ds(0, EDGE_CHUNK)]

    @pl.loop(0, EDGE_NCH)
    def _(ci):
        pltpu.async_copy(ones_src, deg_sp.at[col2d_v.at[ci]], sem_d, add=True)

    @pl.loop(0, EDGE_NCH)
    def _(ci):
        pltpu.make_async_copy(ones_src, deg_sp.at[col2d_v.at[0]], sem_d).wait()

    plsc.subcore_barrier()

    r0 = s * ROWS_PER_SCTILE
    pltpu.sync_copy(deg_sp.at[pl.ds(r0, ROWS_PER_SCTILE)], z_v)
    pltpu.sync_copy(z_v, degp_out.at[pl.ds(c * N_PAD + r0, ROWS_PER_SCTILE)])


_k0 = pl.kernel(
    _k0_body,
    out_type=jax.ShapeDtypeStruct((NC * N_PAD,), jnp.float32),
    mesh=_mesh,
    compiler_params=_cp,
    scratch_types=[
        pltpu.VMEM_SHARED((N_PAD,), jnp.float32),
        pltpu.VMEM((128,), jnp.float32),
        pltpu.VMEM((EDGE_NCH, EDGE_CHUNK), jnp.int32),
        pltpu.VMEM((ROWS_PER_SCTILE,), jnp.float32),
        pltpu.SemaphoreType.DMA,
    ],
)


def _k1_body(fused, *refs):
    if fused:
        (part_in, dis_hbm, bias_hbm, row_hbm, col_hbm, part_out, src_hbm,
         acc_sp,
         row_i0, row_i1, row_i2, row_i3,
         col_i0, col_i1, col_i2, col_i3,
         g0, g1, z_v, dis_v, bias_v,
         sem_i0, sem_i1, sem_i2, sem_i3, sem_g0, sem_g1, sem_s0, sem_s1) = refs
    else:
        (src_hbm, row_hbm, col_hbm, part_out,
         acc_sp,
         row_i0, row_i1, row_i2, row_i3,
         col_i0, col_i1, col_i2, col_i3,
         g0, g1, z_v,
         sem_i0, sem_i1, sem_i2, sem_i3, sem_g0, sem_g1, sem_s0, sem_s1) = refs
    c = lax.axis_index("c")
    s = lax.axis_index("s")
    w = s * NC + c
    ebase = w * (E_TOTAL // NW)

    gbuf = (g0, g1)
    sg = (sem_g0, sem_g1)
    ss = (sem_s0, sem_s1)
    si = (sem_i0, sem_i1, sem_i2, sem_i3)
    rows = (row_i0, row_i1, row_i2, row_i3)
    cols = (col_i0, col_i1, col_i2, col_i3)

    def issue_i(ci, isl):
        e0 = pl.multiple_of(ebase + ci * EDGE_CHUNK, 8)
        pltpu.async_copy(row_hbm.at[pl.ds(e0, EDGE_CHUNK)], rows[isl], si[isl])
        pltpu.async_copy(col_hbm.at[pl.ds(e0, EDGE_CHUNK)], cols[isl], si[isl])

    def wait_i(isl):
        pltpu.make_async_copy(row_hbm.at[pl.ds(0, EDGE_CHUNK)], rows[isl], si[isl]).wait()
        pltpu.make_async_copy(col_hbm.at[pl.ds(0, EDGE_CHUNK)], cols[isl], si[isl]).wait()

    def issue_g(gsl, isl):
        pltpu.async_copy(src_hbm.at[rows[isl]], gbuf[gsl], sg[gsl])

    def wait_g(gsl):
        pltpu.make_async_copy(src_hbm.at[rows[0]], gbuf[gsl], sg[gsl]).wait()

    def issue_s(gsl, isl):
        pltpu.async_copy(gbuf[gsl], acc_sp.at[cols[isl]], ss[gsl], add=True)

    def wait_s(gsl):
        pltpu.make_async_copy(gbuf[gsl], acc_sp.at[cols[0]], ss[gsl]).wait()

    issue_i(0, 3)
    issue_i(1, 0)
    issue_i(2, 1)
    issue_i(3, 2)

    if fused:
        rbase = s * ROWS_PER_SCTILE
        pltpu.sync_copy(dis_hbm.at[pl.ds(rbase, ROWS_PER_SCTILE)], dis_v)
        pltpu.sync_copy(bias_hbm, bias_v)

        @pl.loop(0, ROWS_PER_SCTILE // 64)
        def _(ch):
            r0 = rbase + ch * 64
            pltpu.sync_copy(part_in.at[pl.ds(r0, 64)], z_v.at[pl.ds(0, 64)])
            pltpu.sync_copy(part_in.at[pl.ds(N_PAD + r0, 64)], z_v.at[pl.ds(64, 64)])

            @pl.loop(0, 64 // LANES)
            def _(gq):
                dv = dis_v[pl.ds(ch * 64 + gq * LANES, LANES)]
                for r in range(LANES):
                    sv = dv[r]
                    rr = gq * LANES + r
                    for j in range(D // LANES):
                        sl = pl.ds(j * LANES, LANES)
                        a = z_v[rr, sl] + z_v[64 + rr, sl]
                        z_v[rr, sl] = (sv * sv) * a + sv * bias_v[sl]

            pltpu.sync_copy(z_v.at[pl.ds(0, 64)], src_hbm.at[pl.ds(r0, 64)])

    @pl.loop(0, 128)
    def _(r):
        for j in range(D // LANES):
            z_v[r, pl.ds(j * LANES, LANES)] = _zero16()

    for q in range(ROWS_PER_SCTILE // 128):
        pltpu.sync_copy(z_v, acc_sp.at[pl.ds(s * ROWS_PER_SCTILE + q * 128, 128)])
    plsc.subcore_barrier()

    wait_i(3)
    issue_g(1, 3)
    wait_g(1)
    issue_s(1, 3)
    wait_i(0)
    issue_g(0, 0)

    def quad(q0, last):
        wait_g(0)
        issue_s(0, 0)
        wait_s(1)
        issue_i(q0 + 3, 3)
        wait_i(1)
        issue_g(1, 1)
        wait_g(1)
        issue_s(1, 1)
        wait_s(0)
        if not last:
            issue_i(q0 + 4, 0)
        wait_i(2)
        issue_g(0, 2)
        wait_g(0)
        issue_s(0, 2)
        wait_s(1)
        if not last:
            issue_i(q0 + 5, 1)
        wait_i(3)
        issue_g(1, 3)
        wait_s(0)
        if not last:
            issue_i(q0 + 6, 2)
        wait_g(1)
        issue_s(1, 3)
        if not last:
            wait_i(0)
            issue_g(0, 0)
        else:
            wait_s(1)

    @pl.loop(0, NQUAD - 1)
    def _(t):
        quad(4 * t + 1, last=False)

    quad(4 * (NQUAD - 1) + 1, last=True)
    plsc.subcore_barrier()

    for q in range(ROWS_PER_SCTILE // 128):
        r0 = s * ROWS_PER_SCTILE + q * 128
        pltpu.sync_copy(acc_sp.at[pl.ds(r0, 128)], z_v)
        pltpu.sync_copy(z_v, part_out.at[pl.ds(c * N_PAD + r0, 128)])


_k1 = pl.kernel(
    lambda *refs: _k1_body(False, *refs),
    out_type=jax.ShapeDtypeStruct((NC * N_PAD, D), jnp.float32),
    mesh=_mesh,
    compiler_params=_cp,
    scratch_types=(
        [pltpu.VMEM_SHARED((N_PAD, D), jnp.float32)]
        + [pltpu.VMEM((EDGE_CHUNK,), jnp.int32)] * 8
        + [pltpu.VMEM((EDGE_CHUNK, D), jnp.float32)] * 2
        + [pltpu.VMEM((128, D), jnp.float32)]
        + [pltpu.SemaphoreType.DMA] * 8
    ),
)

_k1f = pl.kernel(
    lambda *refs: _k1_body(True, *refs),
    out_type=[
        jax.ShapeDtypeStruct((NC * N_PAD, D), jnp.float32),
        jax.ShapeDtypeStruct((N_PAD, D), jnp.float32),
    ],
    mesh=_mesh,
    compiler_params=_cp,
    scratch_types=(
        [pltpu.VMEM_SHARED((N_PAD, D), jnp.float32)]
        + [pltpu.VMEM((EDGE_CHUNK,), jnp.int32)] * 8
        + [pltpu.VMEM((EDGE_CHUNK, D), jnp.float32)] * 2
        + [pltpu.VMEM((128, D), jnp.float32)]
        + [pltpu.VMEM((ROWS_PER_SCTILE,), jnp.float32)]
        + [pltpu.VMEM((D,), jnp.float32)]
        + [pltpu.SemaphoreType.DMA] * 8
    ),
)


TC_BLK = 256
TC_GRID = N_PAD // TC_BLK


def _tc_prep_body(degp_ref, deg1_ref, x_ref, dis_ref, tmp_ref):
    d = degp_ref[...] + deg1_ref[...]
    s = jnp.where(d > 0.5, jax.lax.rsqrt(d), 0.0)
    dis_ref[...] = s
    tmp_ref[...] = s * x_ref[...]


def _tc_prep(degp, xp):
    return pl.pallas_call(
        _tc_prep_body,
        grid=(TC_GRID,),
        in_specs=[
            pl.BlockSpec((TC_BLK, 1), lambda i: (i, 0)),
            pl.BlockSpec((TC_BLK, 1), lambda i: (i + TC_GRID, 0)),
            pl.BlockSpec((TC_BLK, D), lambda i: (i, 0)),
        ],
        out_specs=[
            pl.BlockSpec((TC_BLK, 1), lambda i: (i, 0)),
            pl.BlockSpec((TC_BLK, D), lambda i: (i, 0)),
        ],
        out_shape=[
            jax.ShapeDtypeStruct((N_PAD, 1), jnp.float32),
            jax.ShapeDtypeStruct((N_PAD, D), jnp.float32),
        ],
    )(degp, degp, xp)


def _tc_combine_body(scale_out, p0_ref, p1_ref, dis_ref, b_ref, o_ref):
    s = dis_ref[...]
    a = p0_ref[...] + p1_ref[...]
    if scale_out:
        o_ref[...] = (s * s) * a + s * b_ref[...]
    else:
        o_ref[...] = s * a + b_ref[...]


def _tc_combine(part, dis2d, bias2d, scale_out):
    return pl.pallas_call(
        lambda *a: _tc_combine_body(scale_out, *a),
        grid=(TC_GRID,),
        in_specs=[
            pl.BlockSpec((TC_BLK, D), lambda i: (i, 0)),
            pl.BlockSpec((TC_BLK, D), lambda i: (i + TC_GRID, 0)),
            pl.BlockSpec((TC_BLK, 1), lambda i: (i, 0)),
            pl.BlockSpec((1, D), lambda i: (0, 0)),
        ],
        out_specs=pl.BlockSpec((TC_BLK, D), lambda i: (i, 0)),
        out_shape=jax.ShapeDtypeStruct((N_PAD, D), jnp.float32),
    )(part, part, dis2d, bias2d)


def kernel(x, edge_index, prior_index, bias_0, bias_1):
    ei = jnp.concatenate([edge_index, prior_index], axis=1)
    row = ei[0]
    col = ei[1]
    n = x.shape[0]
    col3 = col.reshape(NW, EDGE_NCH, EDGE_CHUNK)
    xp = jnp.zeros((N_PAD, D), jnp.float32).at[:n].set(x)
    b0 = bias_0.reshape(1, D)
    b1 = bias_1.reshape(1, D)

    degp = _k0(col3).reshape(NC * N_PAD, 1)
    dis2d, tmp0 = _tc_prep(degp, xp)
    part1 = _k1(tmp0, row, col)
    part2, _ = _k1f(part1, dis2d.reshape(N_PAD), bias_0, row, col)
    out = _tc_combine(part2, dis2d, b1, False)
    return out[:n]


# speedup vs baseline: 1.1107x; 1.1107x over previous
import dataclasses

import jax
import jax.numpy as jnp
from jax import lax
from jax.experimental import pallas as pl
from jax.experimental.pallas import tpu as pltpu
from jax.experimental.pallas import tpu_sc as plsc

_cp = pltpu.CompilerParams()
if "needs_layout_passes" in pltpu.CompilerParams.__dataclass_fields__:
    _cp = dataclasses.replace(_cp, needs_layout_passes=False)

NC = 2
NS = 16
NW = NC * NS
LANES = 16

N_NODES = 10000
N_PAD = 10240
D = 128
E_TOTAL = 480000

DEG_CHUNK = 120
DEG_NCH = E_TOTAL // NS // DEG_CHUNK
EDGE_CHUNK = 120
EDGE_NCH = E_TOTAL // NW // EDGE_CHUNK
NQUAD = (EDGE_NCH - 1) // 4

ROWS_PER_TILE = N_PAD // NW
ROWS_PER_SCTILE = N_PAD // NS

_mesh = plsc.VectorSubcoreMesh(core_axis_name="c", subcore_axis_name="s")


def _zero16():
    return jnp.zeros((LANES,), jnp.float32)


def _fast_rsqrt(d):
    bits = plsc.bitcast(d, jnp.int32)
    bits = jnp.int32(0x5F3759DF) - (bits >> 1)
    r = plsc.bitcast(bits, jnp.float32)
    for _ in range(3):
        r = r * (1.5 - 0.5 * d * r * r)
    return jnp.where(d > 0.5, r, 0.0)


def _k0_body(col3_hbm, x_hbm, dis_out, tmp_out,
             deg_sp, ones_v, col2d_v, z_v, deg_v, dis_v, x_v, sem_d):
    c = lax.axis_index("c")
    s = lax.axis_index("s")
    w = s * NC + c

    pltpu.sync_copy(col3_hbm.at[s], col2d_v)

    @pl.loop(0, ROWS_PER_SCTILE // LANES)
    def _(i):
        z_v[pl.ds(i * LANES, LANES)] = _zero16()

    for i in range(128 // LANES):
        ones_v[pl.ds(i * LANES, LANES)] = jnp.ones((LANES,), jnp.float32)

    pltpu.sync_copy(z_v, deg_sp.at[pl.ds(s * ROWS_PER_SCTILE, ROWS_PER_SCTILE)])
    plsc.subcore_barrier()

    ones_src = ones_v.at[pl.ds(0, DEG_CHUNK)]

    @pl.loop(0, DEG_NCH)
    def _(ci):
        pltpu.async_copy(ones_src, deg_sp.at[col2d_v.at[ci]], sem_d, add=True)

    @pl.loop(0, DEG_NCH)
    def _(ci):
        pltpu.make_async_copy(ones_src, deg_sp.at[col2d_v.at[0]], sem_d).wait()

    plsc.subcore_barrier()

    nbase = w * ROWS_PER_TILE
    pltpu.sync_copy(deg_sp.at[pl.ds(nbase, ROWS_PER_TILE)], deg_v)

    @pl.loop(0, ROWS_PER_TILE // LANES)
    def _(i):
        d = deg_v[pl.ds(i * LANES, LANES)]
        dis_v[pl.ds(i * LANES, LANES)] = _fast_rsqrt(d)

    pltpu.sync_copy(dis_v, dis_out.at[pl.ds(nbase, ROWS_PER_TILE)])

    @pl.loop(0, ROWS_PER_TILE // 80)
    def _(ch):
        r0 = nbase + ch * 80
        pltpu.sync_copy(x_hbm.at[pl.ds(r0, 80)], x_v)

        @pl.loop(0, 80 // LANES)
        def _(g):
            dv = dis_v[pl.ds(ch * 80 + g * LANES, LANES)]
            for r in range(LANES):
                sv = dv[r]
                row = g * LANES + r
                for j in range(D // LANES):
                    sl = pl.ds(j * LANES, LANES)
                    x_v[row, sl] = x_v[row, sl] * sv

        pltpu.sync_copy(x_v, tmp_out.at[pl.ds(r0, 80)])


_k0 = pl.kernel(
    _k0_body,
    out_type=[
        jax.ShapeDtypeStruct((N_PAD,), jnp.float32),
        jax.ShapeDtypeStruct((N_PAD, D), jnp.float32),
    ],
    mesh=_mesh,
    compiler_params=_cp,
    scratch_types=[
        pltpu.VMEM_SHARED((N_PAD,), jnp.float32),
        pltpu.VMEM((128,), jnp.float32),
        pltpu.VMEM((DEG_NCH, DEG_CHUNK), jnp.int32),
        pltpu.VMEM((ROWS_PER_SCTILE,), jnp.float32),
        pltpu.VMEM((ROWS_PER_TILE,), jnp.float32),
        pltpu.VMEM((ROWS_PER_TILE,), jnp.float32),
        pltpu.VMEM((80, D), jnp.float32),
        pltpu.SemaphoreType.DMA,
    ],
)


def _k1_body(src_hbm, row_hbm, col_hbm, part_out,
             acc_sp,
             row_i0, row_i1, row_i2, row_i3,
             col_i0, col_i1, col_i2, col_i3,
             g0, g1, z_v,
             sem_i0, sem_i1, sem_i2, sem_i3, sem_g0, sem_g1, sem_s0, sem_s1):
    c = lax.axis_index("c")
    s = lax.axis_index("s")
    w = s * NC + c
    ebase = w * (E_TOTAL // NW)

    gbuf = (g0, g1)
    sg = (sem_g0, sem_g1)
    ss = (sem_s0, sem_s1)
    si = (sem_i0, sem_i1, sem_i2, sem_i3)
    rows = (row_i0, row_i1, row_i2, row_i3)
    cols = (col_i0, col_i1, col_i2, col_i3)

    def issue_i(ci, isl):
        e0 = pl.multiple_of(ebase + ci * EDGE_CHUNK, 8)
        pltpu.async_copy(row_hbm.at[pl.ds(e0, EDGE_CHUNK)], rows[isl], si[isl])
        pltpu.async_copy(col_hbm.at[pl.ds(e0, EDGE_CHUNK)], cols[isl], si[isl])

    def wait_i(isl):
        pltpu.make_async_copy(row_hbm.at[pl.ds(0, EDGE_CHUNK)], rows[isl], si[isl]).wait()
        pltpu.make_async_copy(col_hbm.at[pl.ds(0, EDGE_CHUNK)], cols[isl], si[isl]).wait()

    def issue_g(gsl, isl):
        pltpu.async_copy(src_hbm.at[rows[isl]], gbuf[gsl], sg[gsl])

    def wait_g(gsl):
        pltpu.make_async_copy(src_hbm.at[rows[0]], gbuf[gsl], sg[gsl]).wait()

    def issue_s(gsl, isl):
        pltpu.async_copy(gbuf[gsl], acc_sp.at[cols[isl]], ss[gsl], add=True)

    def wait_s(gsl):
        pltpu.make_async_copy(gbuf[gsl], acc_sp.at[cols[0]], ss[gsl]).wait()

    issue_i(0, 3)
    issue_i(1, 0)
    issue_i(2, 1)
    issue_i(3, 2)

    @pl.loop(0, 128)
    def _(r):
        for j in range(D // LANES):
            z_v[r, pl.ds(j * LANES, LANES)] = _zero16()

    for q in range(ROWS_PER_SCTILE // 128):
        pltpu.sync_copy(z_v, acc_sp.at[pl.ds(s * ROWS_PER_SCTILE + q * 128, 128)])
    plsc.subcore_barrier()

    wait_i(3)
    issue_g(1, 3)
    wait_g(1)
    issue_s(1, 3)
    wait_i(0)
    issue_g(0, 0)

    def quad(q0, last):
        wait_g(0)
        issue_s(0, 0)
        wait_s(1)
        issue_i(q0 + 3, 3)
        wait_i(1)
        issue_g(1, 1)
        wait_g(1)
        issue_s(1, 1)
        wait_s(0)
        if not last:
            issue_i(q0 + 4, 0)
        wait_i(2)
        issue_g(0, 2)
        wait_g(0)
        issue_s(0, 2)
        wait_s(1)
        if not last:
            issue_i(q0 + 5, 1)
        wait_i(3)
        issue_g(1, 3)
        wait_s(0)
        if not last:
            issue_i(q0 + 6, 2)
        wait_g(1)
        issue_s(1, 3)
        if not last:
            wait_i(0)
            issue_g(0, 0)
        else:
            wait_s(1)

    @pl.loop(0, NQUAD - 1)
    def _(t):
        quad(4 * t + 1, last=False)

    quad(4 * (NQUAD - 1) + 1, last=True)
    plsc.subcore_barrier()

    for q in range(ROWS_PER_SCTILE // 128):
        r0 = s * ROWS_PER_SCTILE + q * 128
        pltpu.sync_copy(acc_sp.at[pl.ds(r0, 128)], z_v)
        pltpu.sync_copy(z_v, part_out.at[pl.ds(c * N_PAD + r0, 128)])


_k1 = pl.kernel(
    _k1_body,
    out_type=jax.ShapeDtypeStruct((NC * N_PAD, D), jnp.float32),
    mesh=_mesh,
    compiler_params=_cp,
    scratch_types=(
        [pltpu.VMEM_SHARED((N_PAD, D), jnp.float32)]
        + [pltpu.VMEM((EDGE_CHUNK,), jnp.int32)] * 8
        + [pltpu.VMEM((EDGE_CHUNK, D), jnp.float32)] * 2
        + [pltpu.VMEM((128, D), jnp.float32)]
        + [pltpu.SemaphoreType.DMA] * 8
    ),
)

def _k2_body(scale_out, part_hbm, dis_hbm, bias_hbm, o_hbm,
             p0_v, p1_v, dis_v, bias_v):
    c = lax.axis_index("c")
    s = lax.axis_index("s")
    w = s * NC + c
    nbase = w * ROWS_PER_TILE

    pltpu.sync_copy(dis_hbm.at[pl.ds(nbase, ROWS_PER_TILE)], dis_v)
    pltpu.sync_copy(bias_hbm, bias_v)

    @pl.loop(0, ROWS_PER_TILE // 80)
    def _(ch):
        r0 = nbase + ch * 80
        pltpu.sync_copy(part_hbm.at[pl.ds(r0, 80)], p0_v)
        pltpu.sync_copy(part_hbm.at[pl.ds(N_PAD + r0, 80)], p1_v)

        @pl.loop(0, 80 // LANES)
        def _(g):
            dv = dis_v[pl.ds(ch * 80 + g * LANES, LANES)]
            for r in range(LANES):
                sv = dv[r]
                row = g * LANES + r
                for j in range(D // LANES):
                    sl = pl.ds(j * LANES, LANES)
                    a = p0_v[row, sl] + p1_v[row, sl]
                    if scale_out:
                        p0_v[row, sl] = (sv * sv) * a + sv * bias_v[sl]
                    else:
                        p0_v[row, sl] = sv * a + bias_v[sl]

        pltpu.sync_copy(p0_v, o_hbm.at[pl.ds(r0, 80)])


def _make_k2(scale_out):
    return pl.kernel(
        lambda *args: _k2_body(scale_out, *args),
        out_type=jax.ShapeDtypeStruct((N_PAD, D), jnp.float32),
        mesh=_mesh,
        compiler_params=_cp,
        scratch_types=[
            pltpu.VMEM((80, D), jnp.float32),
            pltpu.VMEM((80, D), jnp.float32),
            pltpu.VMEM((ROWS_PER_TILE,), jnp.float32),
            pltpu.VMEM((D,), jnp.float32),
        ],
    )


_k2_mid = _make_k2(True)
_k2_final = _make_k2(False)


def kernel(x, edge_index, prior_index, bias_0, bias_1):
    ei = jnp.concatenate([edge_index, prior_index], axis=1)
    row = ei[0]
    col = ei[1]
    n = x.shape[0]
    col3_deg = col.reshape(NS, DEG_NCH, DEG_CHUNK)
    xp = jnp.zeros((N_PAD, D), jnp.float32).at[:n].set(x)

    dis, tmp0 = _k0(col3_deg, xp)
    part1 = _k1(tmp0, row, col)
    tmp1 = _k2_mid(part1, dis, bias_0)
    part2 = _k1(tmp1, row, col)
    out = _k2_final(part2, dis, bias_1)
    return out[:n]
